# Initial kernel scaffold; baseline (speedup 1.0000x reference)
import jax
import jax.numpy as jnp
from jax.experimental import pallas as pl


def kernel(x, edge_index, W1, b1, W2, b2):
    raise NotImplementedError("write your pallas kernel here")


# speedup vs baseline: 11.0328x; 11.0328x over previous
import functools

import jax
import jax.numpy as jnp
from jax import lax
from jax.experimental import pallas as pl
from jax.experimental.pallas import tpu as pltpu
from jax.experimental.pallas import tpu_sc as plsc

N = 10000
NP = 10240
E = 160000
NC, NS = 2, 16
NW = NC * NS
BLK = 128
EPT = 5120
EP = NW * EPT
NBLK = EPT // BLK
RPT = NP // NS

_mesh = plsc.VectorSubcoreMesh(core_axis_name="c", subcore_axis_name="s")
_sc_params = pltpu.CompilerParams(use_tc_tiling_on_sc=False)


@functools.partial(
    pl.kernel,
    out_type=jax.ShapeDtypeStruct((NC, NP), jnp.float32),
    mesh=_mesh,
    scratch_types=[
        pltpu.VMEM((BLK,), jnp.int32),
        pltpu.VMEM((BLK,), jnp.float32),
        pltpu.VMEM_SHARED((NP,), jnp.float32),
    ],
    compiler_params=_sc_params,
)
def _deg_kernel(dst_hbm, zeros_hbm, out_hbm, idx_v, ones_v, deg_sh):
    cid = lax.axis_index("c")
    sid = lax.axis_index("s")
    wid = cid * NS + sid
    for j in range(BLK // 16):
        ones_v[pl.ds(16 * j, 16)] = jnp.full((16,), 1.0, jnp.float32)
    row0 = sid * RPT
    pltpu.sync_copy(zeros_hbm.at[pl.ds(row0, RPT)], deg_sh.at[pl.ds(row0, RPT)])
    plsc.subcore_barrier()
    base = wid * EPT

    def body(i, carry):
        off = pl.multiple_of(base + i * BLK, BLK)
        pltpu.sync_copy(dst_hbm.at[pl.ds(off, BLK)], idx_v)
        pltpu.sync_copy(ones_v, deg_sh.at[idx_v], add=True)
        return carry

    lax.fori_loop(0, NBLK, body, 0)
    plsc.subcore_barrier()
    pltpu.sync_copy(deg_sh.at[pl.ds(row0, RPT)],
                    out_hbm.at[cid, pl.ds(row0, RPT)])


def _make_agg(width):
    @functools.partial(
        pl.kernel,
        out_type=jax.ShapeDtypeStruct((NC, NP, width), jnp.float32),
        mesh=_mesh,
        scratch_types=[
            pltpu.VMEM((BLK,), jnp.int32),
            pltpu.VMEM((BLK,), jnp.int32),
            pltpu.VMEM((BLK, width), jnp.float32),
            pltpu.SemaphoreType.DMA,
            pltpu.VMEM_SHARED((NP, width), jnp.float32),
        ],
        name=f"gcn_agg{width}",
        compiler_params=_sc_params,
    )
    def agg(table_hbm, src_hbm, dst_hbm, zeros_hbm, out_hbm,
            s_v, d_v, rows_v, sem, acc_sh):
        cid = lax.axis_index("c")
        sid = lax.axis_index("s")
        wid = cid * NS + sid
        row0 = sid * RPT
        pltpu.sync_copy(zeros_hbm.at[pl.ds(row0, RPT)],
                        acc_sh.at[pl.ds(row0, RPT)])
        plsc.subcore_barrier()
        base = wid * EPT

        def body(i, carry):
            off = pl.multiple_of(base + i * BLK, BLK)
            pltpu.sync_copy(src_hbm.at[pl.ds(off, BLK)], s_v)
            pltpu.sync_copy(dst_hbm.at[pl.ds(off, BLK)], d_v)
            pltpu.async_copy(table_hbm.at[s_v], rows_v, sem).wait()
            pltpu.sync_copy(rows_v, acc_sh.at[d_v], add=True)
            return carry

        lax.fori_loop(0, NBLK, body, 0)
        plsc.subcore_barrier()
        pltpu.sync_copy(acc_sh.at[pl.ds(row0, RPT)],
                        out_hbm.at[cid, pl.ds(row0, RPT)])

    return agg


_agg64 = _make_agg(64)
_agg16 = _make_agg(16)


_RB = 2048


def _mm_scale_body(x_ref, w_ref, d0_ref, d1_ref, xws_ref, dis_ref):
    deg = d0_ref[...] + d1_ref[...] + 1.0
    dis = lax.rsqrt(deg)
    xw = jnp.dot(x_ref[...], w_ref[...], preferred_element_type=jnp.float32)
    xws_ref[...] = xw * dis
    dis_ref[...] = dis


_mm_scale = pl.pallas_call(
    _mm_scale_body,
    grid=(NP // _RB,),
    in_specs=[
        pl.BlockSpec((_RB, 256), lambda i: (i, 0)),
        pl.BlockSpec((256, 64), lambda i: (0, 0)),
        pl.BlockSpec((_RB, 1), lambda i: (i, 0)),
        pl.BlockSpec((_RB, 1), lambda i: (i, 0)),
    ],
    out_specs=[
        pl.BlockSpec((_RB, 64), lambda i: (i, 0)),
        pl.BlockSpec((_RB, 1), lambda i: (i, 0)),
    ],
    out_shape=[
        jax.ShapeDtypeStruct((NP, 64), jnp.float32),
        jax.ShapeDtypeStruct((NP, 1), jnp.float32),
    ],
)


def _mid_body(a0_ref, a1_ref, xws_ref, dis_ref, b1_ref, w2_ref, out_ref):
    d = dis_ref[...]
    pre = (a0_ref[...] + a1_ref[...] + xws_ref[...]) * d + b1_ref[...]
    h = jnp.maximum(pre, 0.0)
    out_ref[...] = jnp.dot(h, w2_ref[...],
                           preferred_element_type=jnp.float32) * d


_mid = pl.pallas_call(
    _mid_body,
    grid=(NP // _RB,),
    in_specs=[
        pl.BlockSpec((_RB, 64), lambda i: (i, 0)),
        pl.BlockSpec((_RB, 64), lambda i: (i, 0)),
        pl.BlockSpec((_RB, 64), lambda i: (i, 0)),
        pl.BlockSpec((_RB, 1), lambda i: (i, 0)),
        pl.BlockSpec((1, 64), lambda i: (0, 0)),
        pl.BlockSpec((64, 16), lambda i: (0, 0)),
    ],
    out_specs=pl.BlockSpec((_RB, 16), lambda i: (i, 0)),
    out_shape=jax.ShapeDtypeStruct((NP, 16), jnp.float32),
)


def _final_body(q0_ref, q1_ref, hw_ref, dis_ref, b2_ref, out_ref):
    logits = (q0_ref[...] + q1_ref[...] + hw_ref[...]) * dis_ref[...] \
        + b2_ref[...]
    col = lax.broadcasted_iota(jnp.int32, logits.shape, 1)
    valid = col < 5
    masked = jnp.where(valid, logits, -jnp.inf)
    m = jnp.max(masked, axis=1, keepdims=True)
    e = jnp.where(valid, jnp.exp(logits - m), 0.0)
    lse = jnp.log(jnp.sum(e, axis=1, keepdims=True))
    out_ref[...] = logits - m - lse


_final = pl.pallas_call(
    _final_body,
    grid=(NP // _RB,),
    in_specs=[
        pl.BlockSpec((_RB, 16), lambda i: (i, 0)),
        pl.BlockSpec((_RB, 16), lambda i: (i, 0)),
        pl.BlockSpec((_RB, 16), lambda i: (i, 0)),
        pl.BlockSpec((_RB, 1), lambda i: (i, 0)),
        pl.BlockSpec((1, 16), lambda i: (0, 0)),
    ],
    out_specs=pl.BlockSpec((_RB, 16), lambda i: (i, 0)),
    out_shape=jax.ShapeDtypeStruct((NP, 16), jnp.float32),
)


def kernel(x, edge_index, W1, b1, W2, b2):
    ei = edge_index.astype(jnp.int32)
    pad = jnp.full((EP - E,), N, jnp.int32)
    src = jnp.concatenate([ei[0], pad])
    dst = jnp.concatenate([ei[1], pad])
    xp = jnp.pad(x, ((0, NP - N), (0, 0)))
    z1 = jnp.zeros((NP,), jnp.float32)
    z64 = jnp.zeros((NP, 64), jnp.float32)
    z16 = jnp.zeros((NP, 16), jnp.float32)
    w2p = jnp.pad(W2, ((0, 0), (0, 16 - W2.shape[1])))
    b1r = b1.reshape(1, 64)
    b2p = jnp.pad(b2, (0, 16 - b2.shape[0])).reshape(1, 16)

    deg = _deg_kernel(dst, z1)
    d0 = deg[0].reshape(NP, 1)
    d1 = deg[1].reshape(NP, 1)
    xws, dis = _mm_scale(xp, W1, d0, d1)
    a = _agg64(xws, src, dst, z64)
    hw2s = _mid(a[0], a[1], xws, dis, b1r, w2p)
    q = _agg16(hw2s, src, dst, z16)
    outp = _final(q[0], q[1], hw2s, dis, b2p)
    return outp[:N, :5]


# speedup vs baseline: 15.7267x; 1.4255x over previous
import functools

import jax
import jax.numpy as jnp
from jax import lax
from jax.experimental import pallas as pl
from jax.experimental.pallas import tpu as pltpu
from jax.experimental.pallas import tpu_sc as plsc

N = 10000
NP = 10240
E = 160000
NC, NS = 2, 16
NW = NC * NS
BLK = 128
EPT = 5120
EP = NW * EPT
NBLK = EPT // BLK
RPT = NP // NS

_mesh = plsc.VectorSubcoreMesh(core_axis_name="c", subcore_axis_name="s")
_sc_params = pltpu.CompilerParams(use_tc_tiling_on_sc=False)


@functools.partial(
    pl.kernel,
    out_type=jax.ShapeDtypeStruct((NC, NP), jnp.float32),
    mesh=_mesh,
    scratch_types=[
        pltpu.VMEM((NBLK, BLK), jnp.int32),
        pltpu.VMEM((BLK,), jnp.float32),
        pltpu.VMEM_SHARED((NP,), jnp.float32),
    ],
    compiler_params=_sc_params,
)
def _deg_kernel(dst_hbm, zeros_hbm, out_hbm, idx_v, ones_v, deg_sh):
    cid = lax.axis_index("c")
    sid = lax.axis_index("s")
    wid = cid * NS + sid
    for j in range(BLK // 16):
        ones_v[pl.ds(16 * j, 16)] = jnp.full((16,), 1.0, jnp.float32)
    row0 = sid * RPT
    pltpu.sync_copy(dst_hbm.at[wid], idx_v)
    pltpu.sync_copy(zeros_hbm.at[pl.ds(row0, RPT)], deg_sh.at[pl.ds(row0, RPT)])
    plsc.subcore_barrier()

    def body(i, carry):
        pltpu.sync_copy(ones_v, deg_sh.at[idx_v.at[i]], add=True)
        return carry

    lax.fori_loop(0, NBLK, body, 0)
    plsc.subcore_barrier()
    pltpu.sync_copy(deg_sh.at[pl.ds(row0, RPT)],
                    out_hbm.at[cid, pl.ds(row0, RPT)])


_NB = 4


def _make_agg(width):
    @functools.partial(
        pl.kernel,
        out_type=jax.ShapeDtypeStruct((NC, NP, width), jnp.float32),
        mesh=_mesh,
        scratch_types=[
            pltpu.VMEM((NBLK, BLK), jnp.int32),
            pltpu.VMEM((NBLK, BLK), jnp.int32),
            pltpu.VMEM((_NB, BLK, width), jnp.float32),
            pltpu.SemaphoreType.DMA((_NB,)),
            pltpu.VMEM_SHARED((NP, width), jnp.float32),
        ],
        name=f"gcn_agg{width}",
        compiler_params=_sc_params,
    )
    def agg(table_hbm, src_hbm, dst_hbm, zeros_hbm, out_hbm,
            s_v, d_v, rows_v, sems, acc_sh):
        cid = lax.axis_index("c")
        sid = lax.axis_index("s")
        wid = cid * NS + sid
        row0 = sid * RPT
        pltpu.sync_copy(src_hbm.at[wid], s_v)
        pltpu.sync_copy(dst_hbm.at[wid], d_v)
        pltpu.sync_copy(zeros_hbm.at[pl.ds(row0, RPT)],
                        acc_sh.at[pl.ds(row0, RPT)])
        plsc.subcore_barrier()

        for k in range(_NB):
            pltpu.async_copy(table_hbm.at[s_v.at[k]], rows_v.at[k],
                             sems.at[k])

        def body(j, carry):
            blk0 = j * _NB
            for k in range(_NB):
                blk = blk0 + k
                pltpu.make_async_copy(table_hbm.at[s_v.at[k]],
                                      rows_v.at[k], sems.at[k]).wait()
                pltpu.sync_copy(rows_v.at[k], acc_sh.at[d_v.at[blk]],
                                add=True)

                @pl.when(blk + _NB < NBLK)
                def _():
                    pltpu.async_copy(table_hbm.at[s_v.at[blk + _NB]],
                                     rows_v.at[k], sems.at[k])
            return carry

        lax.fori_loop(0, NBLK // _NB, body, 0)
        plsc.subcore_barrier()
        pltpu.sync_copy(acc_sh.at[pl.ds(row0, RPT)],
                        out_hbm.at[cid, pl.ds(row0, RPT)])

    return agg


_agg64 = _make_agg(64)
_agg16 = _make_agg(16)


_RB = 2048


def _mm_scale_body(x_ref, w_ref, d0_ref, d1_ref, xws_ref, dis_ref):
    deg = d0_ref[...] + d1_ref[...] + 1.0
    dis = lax.rsqrt(deg)
    xw = jnp.dot(x_ref[...], w_ref[...], preferred_element_type=jnp.float32)
    xws_ref[...] = xw * dis
    dis_ref[...] = dis


_mm_scale = pl.pallas_call(
    _mm_scale_body,
    grid=(NP // _RB,),
    in_specs=[
        pl.BlockSpec((_RB, 256), lambda i: (i, 0)),
        pl.BlockSpec((256, 64), lambda i: (0, 0)),
        pl.BlockSpec((_RB, 1), lambda i: (i, 0)),
        pl.BlockSpec((_RB, 1), lambda i: (i, 0)),
    ],
    out_specs=[
        pl.BlockSpec((_RB, 64), lambda i: (i, 0)),
        pl.BlockSpec((_RB, 1), lambda i: (i, 0)),
    ],
    out_shape=[
        jax.ShapeDtypeStruct((NP, 64), jnp.float32),
        jax.ShapeDtypeStruct((NP, 1), jnp.float32),
    ],
)


def _mid_body(a0_ref, a1_ref, xws_ref, dis_ref, b1_ref, w2_ref, out_ref):
    d = dis_ref[...]
    pre = (a0_ref[...] + a1_ref[...] + xws_ref[...]) * d + b1_ref[...]
    h = jnp.maximum(pre, 0.0)
    out_ref[...] = jnp.dot(h, w2_ref[...],
                           preferred_element_type=jnp.float32) * d


_mid = pl.pallas_call(
    _mid_body,
    grid=(NP // _RB,),
    in_specs=[
        pl.BlockSpec((_RB, 64), lambda i: (i, 0)),
        pl.BlockSpec((_RB, 64), lambda i: (i, 0)),
        pl.BlockSpec((_RB, 64), lambda i: (i, 0)),
        pl.BlockSpec((_RB, 1), lambda i: (i, 0)),
        pl.BlockSpec((1, 64), lambda i: (0, 0)),
        pl.BlockSpec((64, 16), lambda i: (0, 0)),
    ],
    out_specs=pl.BlockSpec((_RB, 16), lambda i: (i, 0)),
    out_shape=jax.ShapeDtypeStruct((NP, 16), jnp.float32),
)


def _final_body(q0_ref, q1_ref, hw_ref, dis_ref, b2_ref, out_ref):
    logits = (q0_ref[...] + q1_ref[...] + hw_ref[...]) * dis_ref[...] \
        + b2_ref[...]
    col = lax.broadcasted_iota(jnp.int32, logits.shape, 1)
    valid = col < 5
    masked = jnp.where(valid, logits, -jnp.inf)
    m = jnp.max(masked, axis=1, keepdims=True)
    e = jnp.where(valid, jnp.exp(logits - m), 0.0)
    lse = jnp.log(jnp.sum(e, axis=1, keepdims=True))
    out_ref[...] = logits - m - lse


_final = pl.pallas_call(
    _final_body,
    grid=(NP // _RB,),
    in_specs=[
        pl.BlockSpec((_RB, 16), lambda i: (i, 0)),
        pl.BlockSpec((_RB, 16), lambda i: (i, 0)),
        pl.BlockSpec((_RB, 16), lambda i: (i, 0)),
        pl.BlockSpec((_RB, 1), lambda i: (i, 0)),
        pl.BlockSpec((1, 16), lambda i: (0, 0)),
    ],
    out_specs=pl.BlockSpec((_RB, 16), lambda i: (i, 0)),
    out_shape=jax.ShapeDtypeStruct((NP, 16), jnp.float32),
)


def kernel(x, edge_index, W1, b1, W2, b2):
    ei = edge_index.astype(jnp.int32)
    pad = jnp.full((EP - E,), N, jnp.int32)
    src = jnp.concatenate([ei[0], pad]).reshape(NW, NBLK, BLK)
    dst = jnp.concatenate([ei[1], pad]).reshape(NW, NBLK, BLK)
    xp = jnp.pad(x, ((0, NP - N), (0, 0)))
    z1 = jnp.zeros((NP,), jnp.float32)
    z64 = jnp.zeros((NP, 64), jnp.float32)
    z16 = jnp.zeros((NP, 16), jnp.float32)
    w2p = jnp.pad(W2, ((0, 0), (0, 16 - W2.shape[1])))
    b1r = b1.reshape(1, 64)
    b2p = jnp.pad(b2, (0, 16 - b2.shape[0])).reshape(1, 16)

    deg = _deg_kernel(dst, z1)
    d0 = deg[0].reshape(NP, 1)
    d1 = deg[1].reshape(NP, 1)
    xws, dis = _mm_scale(xp, W1, d0, d1)
    a = _agg64(xws, src, dst, z64)
    hw2s = _mid(a[0], a[1], xws, dis, b1r, w2p)
    q = _agg16(hw2s, src, dst, z16)
    outp = _final(q[0], q[1], hw2s, dis, b2p)
    return outp[:N, :5]


# speedup vs baseline: 24.8342x; 1.5791x over previous
import functools

import jax
import jax.numpy as jnp
from jax import lax
from jax.experimental import pallas as pl
from jax.experimental.pallas import tpu as pltpu
from jax.experimental.pallas import tpu_sc as plsc

N = 10000
NP = 10240
E = 160000
NC, NS = 2, 16
NW = NC * NS
BLK = 128
EPT = 5120
EP = NW * EPT
NBLK = EPT // BLK
RPT = NP // NS

_mesh = plsc.VectorSubcoreMesh(core_axis_name="c", subcore_axis_name="s")
_sc_params = pltpu.CompilerParams(use_tc_tiling_on_sc=False)


@functools.partial(
    pl.kernel,
    out_type=jax.ShapeDtypeStruct((NC, NP), jnp.float32),
    mesh=_mesh,
    scratch_types=[
        pltpu.VMEM((NBLK, BLK), jnp.int32),
        pltpu.VMEM((BLK,), jnp.float32),
        pltpu.VMEM_SHARED((NP,), jnp.float32),
    ],
    compiler_params=_sc_params,
)
def _deg_kernel(dst_hbm, zeros_hbm, out_hbm, idx_v, ones_v, deg_sh):
    cid = lax.axis_index("c")
    sid = lax.axis_index("s")
    wid = cid * NS + sid
    for j in range(BLK // 16):
        ones_v[pl.ds(16 * j, 16)] = jnp.full((16,), 1.0, jnp.float32)
    row0 = sid * RPT
    pltpu.sync_copy(dst_hbm.at[wid], idx_v)
    pltpu.sync_copy(zeros_hbm.at[pl.ds(row0, RPT)], deg_sh.at[pl.ds(row0, RPT)])
    plsc.subcore_barrier()

    def body(i, carry):
        pltpu.sync_copy(ones_v, deg_sh.at[idx_v.at[i]], add=True)
        return carry

    lax.fori_loop(0, NBLK, body, 0)
    plsc.subcore_barrier()
    pltpu.sync_copy(deg_sh.at[pl.ds(row0, RPT)],
                    out_hbm.at[cid, pl.ds(row0, RPT)])


_NB = 4


def _make_agg(width):
    @functools.partial(
        pl.kernel,
        out_type=jax.ShapeDtypeStruct((NC, NP, width), jnp.float32),
        mesh=_mesh,
        scratch_types=[
            pltpu.VMEM((NBLK, BLK), jnp.int32),
            pltpu.VMEM((NBLK, BLK), jnp.int32),
            pltpu.VMEM((_NB, BLK, width), jnp.float32),
            pltpu.SemaphoreType.DMA((_NB,)),
            pltpu.VMEM_SHARED((NP, width), jnp.float32),
            pltpu.VMEM_SHARED((NP, width), jnp.float32),
        ],
        name=f"gcn_agg{width}",
        compiler_params=_sc_params,
    )
    def agg(table_hbm, src_hbm, dst_hbm, zeros_hbm, out_hbm,
            s_v, d_v, rows_v, sems, acc_sh, table_sh):
        cid = lax.axis_index("c")
        sid = lax.axis_index("s")
        wid = cid * NS + sid
        row0 = sid * RPT
        pltpu.sync_copy(src_hbm.at[wid], s_v)
        pltpu.sync_copy(dst_hbm.at[wid], d_v)
        pltpu.sync_copy(table_hbm.at[pl.ds(row0, RPT)],
                        table_sh.at[pl.ds(row0, RPT)])
        pltpu.sync_copy(zeros_hbm.at[pl.ds(row0, RPT)],
                        acc_sh.at[pl.ds(row0, RPT)])
        plsc.subcore_barrier()

        for k in range(_NB):
            pltpu.async_copy(table_sh.at[s_v.at[k]], rows_v.at[k],
                             sems.at[k])

        def body(j, carry):
            blk0 = j * _NB
            for k in range(_NB):
                blk = blk0 + k
                pltpu.make_async_copy(table_sh.at[s_v.at[k]],
                                      rows_v.at[k], sems.at[k]).wait()
                pltpu.sync_copy(rows_v.at[k], acc_sh.at[d_v.at[blk]],
                                add=True)

                @pl.when(blk + _NB < NBLK)
                def _():
                    pltpu.async_copy(table_sh.at[s_v.at[blk + _NB]],
                                     rows_v.at[k], sems.at[k])
            return carry

        lax.fori_loop(0, NBLK // _NB, body, 0)
        plsc.subcore_barrier()
        pltpu.sync_copy(acc_sh.at[pl.ds(row0, RPT)],
                        out_hbm.at[cid, pl.ds(row0, RPT)])

    return agg


_agg64 = _make_agg(64)
_agg16 = _make_agg(16)


_RB = 2048


def _mm_scale_body(x_ref, w_ref, d0_ref, d1_ref, xws_ref, dis_ref):
    deg = d0_ref[...] + d1_ref[...] + 1.0
    dis = lax.rsqrt(deg)
    xw = jnp.dot(x_ref[...], w_ref[...], preferred_element_type=jnp.float32)
    xws_ref[...] = xw * dis
    dis_ref[...] = dis


_mm_scale = pl.pallas_call(
    _mm_scale_body,
    grid=(NP // _RB,),
    in_specs=[
        pl.BlockSpec((_RB, 256), lambda i: (i, 0)),
        pl.BlockSpec((256, 64), lambda i: (0, 0)),
        pl.BlockSpec((_RB, 1), lambda i: (i, 0)),
        pl.BlockSpec((_RB, 1), lambda i: (i, 0)),
    ],
    out_specs=[
        pl.BlockSpec((_RB, 64), lambda i: (i, 0)),
        pl.BlockSpec((_RB, 1), lambda i: (i, 0)),
    ],
    out_shape=[
        jax.ShapeDtypeStruct((NP, 64), jnp.float32),
        jax.ShapeDtypeStruct((NP, 1), jnp.float32),
    ],
)


def _mid_body(a0_ref, a1_ref, xws_ref, dis_ref, b1_ref, w2_ref, out_ref):
    d = dis_ref[...]
    pre = (a0_ref[...] + a1_ref[...] + xws_ref[...]) * d + b1_ref[...]
    h = jnp.maximum(pre, 0.0)
    out_ref[...] = jnp.dot(h, w2_ref[...],
                           preferred_element_type=jnp.float32) * d


_mid = pl.pallas_call(
    _mid_body,
    grid=(NP // _RB,),
    in_specs=[
        pl.BlockSpec((_RB, 64), lambda i: (i, 0)),
        pl.BlockSpec((_RB, 64), lambda i: (i, 0)),
        pl.BlockSpec((_RB, 64), lambda i: (i, 0)),
        pl.BlockSpec((_RB, 1), lambda i: (i, 0)),
        pl.BlockSpec((1, 64), lambda i: (0, 0)),
        pl.BlockSpec((64, 16), lambda i: (0, 0)),
    ],
    out_specs=pl.BlockSpec((_RB, 16), lambda i: (i, 0)),
    out_shape=jax.ShapeDtypeStruct((NP, 16), jnp.float32),
)


def _final_body(q0_ref, q1_ref, hw_ref, dis_ref, b2_ref, out_ref):
    logits = (q0_ref[...] + q1_ref[...] + hw_ref[...]) * dis_ref[...] \
        + b2_ref[...]
    col = lax.broadcasted_iota(jnp.int32, logits.shape, 1)
    valid = col < 5
    masked = jnp.where(valid, logits, -jnp.inf)
    m = jnp.max(masked, axis=1, keepdims=True)
    e = jnp.where(valid, jnp.exp(logits - m), 0.0)
    lse = jnp.log(jnp.sum(e, axis=1, keepdims=True))
    out_ref[...] = logits - m - lse


_final = pl.pallas_call(
    _final_body,
    grid=(NP // _RB,),
    in_specs=[
        pl.BlockSpec((_RB, 16), lambda i: (i, 0)),
        pl.BlockSpec((_RB, 16), lambda i: (i, 0)),
        pl.BlockSpec((_RB, 16), lambda i: (i, 0)),
        pl.BlockSpec((_RB, 1), lambda i: (i, 0)),
        pl.BlockSpec((1, 16), lambda i: (0, 0)),
    ],
    out_specs=pl.BlockSpec((_RB, 16), lambda i: (i, 0)),
    out_shape=jax.ShapeDtypeStruct((NP, 16), jnp.float32),
)


def kernel(x, edge_index, W1, b1, W2, b2):
    ei = edge_index.astype(jnp.int32)
    pad = jnp.full((EP - E,), N, jnp.int32)
    src = jnp.concatenate([ei[0], pad]).reshape(NW, NBLK, BLK)
    dst = jnp.concatenate([ei[1], pad]).reshape(NW, NBLK, BLK)
    xp = jnp.pad(x, ((0, NP - N), (0, 0)))
    z1 = jnp.zeros((NP,), jnp.float32)
    z64 = jnp.zeros((NP, 64), jnp.float32)
    z16 = jnp.zeros((NP, 16), jnp.float32)
    w2p = jnp.pad(W2, ((0, 0), (0, 16 - W2.shape[1])))
    b1r = b1.reshape(1, 64)
    b2p = jnp.pad(b2, (0, 16 - b2.shape[0])).reshape(1, 16)

    deg = _deg_kernel(dst, z1)
    d0 = deg[0].reshape(NP, 1)
    d1 = deg[1].reshape(NP, 1)
    xws, dis = _mm_scale(xp, W1, d0, d1)
    a = _agg64(xws, src, dst, z64)
    hw2s = _mid(a[0], a[1], xws, dis, b1r, w2p)
    q = _agg16(hw2s, src, dst, z16)
    outp = _final(q[0], q[1], hw2s, dis, b2p)
    return outp[:N, :5]


# speedup vs baseline: 28.9951x; 1.1675x over previous
import functools

import jax
import jax.numpy as jnp
from jax import lax
from jax.experimental import pallas as pl
from jax.experimental.pallas import tpu as pltpu
from jax.experimental.pallas import tpu_sc as plsc

N = 10000
NP = 10240
E = 160000
NC, NS = 2, 16
NW = NC * NS
BLK = 128
EPT = 5120
EP = NW * EPT
NBLK = EPT // BLK
RPT = NP // NS
NPH = NP - N

_mesh = plsc.VectorSubcoreMesh(core_axis_name="c", subcore_axis_name="s")
_sc_params = pltpu.CompilerParams(use_tc_tiling_on_sc=False)


@functools.partial(
    pl.kernel,
    out_type=jax.ShapeDtypeStruct((NC, NP), jnp.float32),
    mesh=_mesh,
    scratch_types=[
        pltpu.VMEM((NBLK, BLK), jnp.int32),
        pltpu.VMEM((BLK,), jnp.float32),
        pltpu.VMEM_SHARED((NP,), jnp.float32),
    ],
    compiler_params=_sc_params,
)
def _deg_kernel(srcdst_hbm, z640_hbm, out_hbm, idx_v, ones_v, deg_sh):
    cid = lax.axis_index("c")
    sid = lax.axis_index("s")
    wid = cid * NS + sid
    for j in range(BLK // 16):
        ones_v[pl.ds(16 * j, 16)] = jnp.full((16,), 1.0, jnp.float32)
    row0 = sid * RPT
    pltpu.sync_copy(srcdst_hbm.at[1, wid], idx_v)
    pltpu.sync_copy(z640_hbm, deg_sh.at[pl.ds(row0, RPT)])
    plsc.subcore_barrier()

    def body(i, carry):
        pltpu.sync_copy(ones_v, deg_sh.at[idx_v.at[i]], add=True)
        return carry

    lax.fori_loop(0, NBLK, body, 0)
    plsc.subcore_barrier()
    pltpu.sync_copy(deg_sh.at[pl.ds(row0, RPT)],
                    out_hbm.at[cid, pl.ds(row0, RPT)])


_NB = 4


def _make_agg(width):
    @functools.partial(
        pl.kernel,
        out_type=jax.ShapeDtypeStruct((NC, NP, width), jnp.float32),
        mesh=_mesh,
        scratch_types=[
            pltpu.VMEM((NBLK, BLK), jnp.int32),
            pltpu.VMEM((NBLK, BLK), jnp.int32),
            pltpu.VMEM((_NB, BLK, width), jnp.float32),
            pltpu.SemaphoreType.DMA((_NB,)),
            pltpu.VMEM_SHARED((NP, width), jnp.float32),
            pltpu.VMEM_SHARED((NP, width), jnp.float32),
        ],
        name=f"gcn_agg{width}",
        compiler_params=_sc_params,
    )
    def agg(table_hbm, srcdst_hbm, z640_hbm, out_hbm,
            s_v, d_v, rows_v, sems, acc_sh, table_sh):
        cid = lax.axis_index("c")
        sid = lax.axis_index("s")
        wid = cid * NS + sid
        row0 = sid * RPT
        pltpu.sync_copy(srcdst_hbm.at[0, wid], s_v)
        pltpu.sync_copy(srcdst_hbm.at[1, wid], d_v)

        @pl.when(sid < NS - 1)
        def _():
            pltpu.sync_copy(table_hbm.at[pl.ds(row0, RPT)],
                            table_sh.at[pl.ds(row0, RPT)])

        @pl.when(sid == NS - 1)
        def _():
            pltpu.sync_copy(table_hbm.at[pl.ds(NS * RPT - RPT, N - (NS - 1) * RPT)],
                            table_sh.at[pl.ds(NS * RPT - RPT, N - (NS - 1) * RPT)])
            pltpu.sync_copy(z640_hbm.at[pl.ds(0, NPH)],
                            table_sh.at[pl.ds(N, NPH)])

        pltpu.sync_copy(z640_hbm, acc_sh.at[pl.ds(row0, RPT)])
        plsc.subcore_barrier()

        for k in range(_NB):
            pltpu.async_copy(table_sh.at[s_v.at[k]], rows_v.at[k],
                             sems.at[k])

        def body(j, carry):
            blk0 = j * _NB
            for k in range(_NB):
                blk = blk0 + k
                pltpu.make_async_copy(table_sh.at[s_v.at[k]],
                                      rows_v.at[k], sems.at[k]).wait()
                pltpu.sync_copy(rows_v.at[k], acc_sh.at[d_v.at[blk]],
                                add=True)

                @pl.when(blk + _NB < NBLK)
                def _():
                    pltpu.async_copy(table_sh.at[s_v.at[blk + _NB]],
                                     rows_v.at[k], sems.at[k])
            return carry

        lax.fori_loop(0, NBLK // _NB, body, 0)
        plsc.subcore_barrier()
        pltpu.sync_copy(acc_sh.at[pl.ds(row0, RPT)],
                        out_hbm.at[cid, pl.ds(row0, RPT)])

    return agg


_agg64 = _make_agg(64)
_agg16 = _make_agg(16)


_RB = 2000


def _mm_body(x_ref, w_ref, out_ref):
    out_ref[...] = jnp.dot(x_ref[...], w_ref[...],
                           preferred_element_type=jnp.float32)


_mm = pl.pallas_call(
    _mm_body,
    grid=(N // _RB,),
    in_specs=[
        pl.BlockSpec((_RB, 256), lambda i: (i, 0)),
        pl.BlockSpec((256, 64), lambda i: (0, 0)),
    ],
    out_specs=pl.BlockSpec((_RB, 64), lambda i: (i, 0)),
    out_shape=jax.ShapeDtypeStruct((N, 64), jnp.float32),
)


def _scale_body(xw_ref, degt_ref, xws_ref, dis_ref):
    d = degt_ref[...]
    deg = d[:, 0:1] + d[:, 1:2] + 1.0
    dis = lax.rsqrt(deg)
    xws_ref[...] = xw_ref[...] * dis
    dis_ref[...] = dis


_scale = pl.pallas_call(
    _scale_body,
    grid=(N // _RB,),
    in_specs=[
        pl.BlockSpec((_RB, 64), lambda i: (i, 0)),
        pl.BlockSpec((_RB, NC), lambda i: (i, 0)),
    ],
    out_specs=[
        pl.BlockSpec((_RB, 64), lambda i: (i, 0)),
        pl.BlockSpec((_RB, 1), lambda i: (i, 0)),
    ],
    out_shape=[
        jax.ShapeDtypeStruct((N, 64), jnp.float32),
        jax.ShapeDtypeStruct((N, 1), jnp.float32),
    ],
)


def _mid_body(a_ref, xws_ref, dis_ref, b1_ref, w2_ref, out_ref):
    d = dis_ref[...]
    pre = (a_ref[0] + a_ref[1] + xws_ref[...]) * d + b1_ref[...]
    h = jnp.maximum(pre, 0.0)
    out_ref[...] = jnp.dot(h, w2_ref[...],
                           preferred_element_type=jnp.float32) * d


_mid = pl.pallas_call(
    _mid_body,
    grid=(N // _RB,),
    in_specs=[
        pl.BlockSpec((NC, _RB, 64), lambda i: (0, i, 0)),
        pl.BlockSpec((_RB, 64), lambda i: (i, 0)),
        pl.BlockSpec((_RB, 1), lambda i: (i, 0)),
        pl.BlockSpec((1, 64), lambda i: (0, 0)),
        pl.BlockSpec((64, 16), lambda i: (0, 0)),
    ],
    out_specs=pl.BlockSpec((_RB, 16), lambda i: (i, 0)),
    out_shape=jax.ShapeDtypeStruct((N, 16), jnp.float32),
)


def _final_body(q_ref, hw_ref, dis_ref, b2_ref, out_ref):
    logits = (q_ref[0] + q_ref[1] + hw_ref[...]) * dis_ref[...] + b2_ref[...]
    col = lax.broadcasted_iota(jnp.int32, logits.shape, 1)
    valid = col < 5
    masked = jnp.where(valid, logits, -jnp.inf)
    m = jnp.max(masked, axis=1, keepdims=True)
    e = jnp.where(valid, jnp.exp(logits - m), 0.0)
    lse = jnp.log(jnp.sum(e, axis=1, keepdims=True))
    out_ref[...] = (logits - m - lse)[:, :5]


_final = pl.pallas_call(
    _final_body,
    grid=(N // _RB,),
    in_specs=[
        pl.BlockSpec((NC, _RB, 16), lambda i: (0, i, 0)),
        pl.BlockSpec((_RB, 16), lambda i: (i, 0)),
        pl.BlockSpec((_RB, 1), lambda i: (i, 0)),
        pl.BlockSpec((1, 16), lambda i: (0, 0)),
    ],
    out_specs=pl.BlockSpec((_RB, 5), lambda i: (i, 0)),
    out_shape=jax.ShapeDtypeStruct((N, 5), jnp.float32),
)


def kernel(x, edge_index, W1, b1, W2, b2):
    ei = edge_index.astype(jnp.int32)
    ph = N + (jnp.arange(EP - E, dtype=jnp.int32) % NPH)
    srcdst = jnp.concatenate([ei, jnp.stack([ph, ph])], axis=1)
    srcdst = srcdst.reshape(2, NW, NBLK, BLK)
    z640 = jnp.zeros((RPT,), jnp.float32)
    z640_64 = jnp.zeros((RPT, 64), jnp.float32)
    z640_16 = jnp.zeros((RPT, 16), jnp.float32)
    w2p = jnp.pad(W2, ((0, 0), (0, 16 - W2.shape[1])))
    b1r = b1.reshape(1, 64)
    b2p = jnp.pad(b2, (0, 16 - b2.shape[0])).reshape(1, 16)

    deg = _deg_kernel(srcdst, z640)
    xw = _mm(x, W1)
    xws, dis = _scale(xw, deg.T)
    a = _agg64(xws, srcdst, z640_64)
    hw2s = _mid(a, xws, dis, b1r, w2p)
    q = _agg16(hw2s, srcdst, z640_16)
    return _final(q, hw2s, dis, b2p)


# speedup vs baseline: 32.5356x; 1.1221x over previous
import functools

import jax
import jax.numpy as jnp
from jax import lax
from jax.experimental import pallas as pl
from jax.experimental.pallas import tpu as pltpu
from jax.experimental.pallas import tpu_sc as plsc

N = 10000
NP = 10240
E = 160000
NC, NS = 2, 16
NW = NC * NS
BLK = 128
EPT = 5120
EP = NW * EPT
NBLK = EPT // BLK
RPT = NP // NS
NPH = NP - N

_mesh = plsc.VectorSubcoreMesh(core_axis_name="c", subcore_axis_name="s")
_sc_params = pltpu.CompilerParams(use_tc_tiling_on_sc=False)


@functools.partial(
    pl.kernel,
    out_type=jax.ShapeDtypeStruct((NC, NP), jnp.float32),
    mesh=_mesh,
    scratch_types=[
        pltpu.VMEM((NBLK, BLK), jnp.int32),
        pltpu.VMEM((BLK,), jnp.float32),
        pltpu.VMEM_SHARED((NP,), jnp.float32),
    ],
    compiler_params=_sc_params,
)
def _deg_kernel(srcdst_hbm, z640_hbm, out_hbm, idx_v, ones_v, deg_sh):
    cid = lax.axis_index("c")
    sid = lax.axis_index("s")
    wid = cid * NS + sid
    for j in range(BLK // 16):
        ones_v[pl.ds(16 * j, 16)] = jnp.full((16,), 1.0, jnp.float32)
    row0 = sid * RPT
    pltpu.sync_copy(srcdst_hbm.at[1, wid], idx_v)
    pltpu.sync_copy(z640_hbm, deg_sh.at[pl.ds(row0, RPT)])
    plsc.subcore_barrier()

    def body(i, carry):
        pltpu.sync_copy(ones_v, deg_sh.at[idx_v.at[i]], add=True)
        return carry

    lax.fori_loop(0, NBLK, body, 0)
    plsc.subcore_barrier()
    pltpu.sync_copy(deg_sh.at[pl.ds(row0, RPT)],
                    out_hbm.at[cid, pl.ds(row0, RPT)])


_NB = 4


def _make_agg(width, dtype):
    @functools.partial(
        pl.kernel,
        out_type=jax.ShapeDtypeStruct((NC, NP, width), dtype),
        mesh=_mesh,
        scratch_types=[
            pltpu.VMEM((NBLK, BLK), jnp.int32),
            pltpu.VMEM((NBLK, BLK), jnp.int32),
            pltpu.VMEM((_NB, BLK, width), dtype),
            pltpu.SemaphoreType.DMA((_NB,)),
            pltpu.VMEM_SHARED((NP, width), dtype),
            pltpu.VMEM_SHARED((NP, width), dtype),
        ],
        name=f"gcn_agg{width}",
        compiler_params=_sc_params,
    )
    def agg(table_hbm, srcdst_hbm, z640_hbm, out_hbm,
            s_v, d_v, rows_v, sems, acc_sh, table_sh):
        cid = lax.axis_index("c")
        sid = lax.axis_index("s")
        wid = cid * NS + sid
        row0 = sid * RPT
        pltpu.sync_copy(srcdst_hbm.at[0, wid], s_v)
        pltpu.sync_copy(srcdst_hbm.at[1, wid], d_v)

        @pl.when(sid < NS - 1)
        def _():
            pltpu.sync_copy(table_hbm.at[pl.ds(row0, RPT)],
                            table_sh.at[pl.ds(row0, RPT)])

        @pl.when(sid == NS - 1)
        def _():
            pltpu.sync_copy(table_hbm.at[pl.ds(NS * RPT - RPT, N - (NS - 1) * RPT)],
                            table_sh.at[pl.ds(NS * RPT - RPT, N - (NS - 1) * RPT)])
            pltpu.sync_copy(z640_hbm.at[pl.ds(0, NPH)],
                            table_sh.at[pl.ds(N, NPH)])

        pltpu.sync_copy(z640_hbm, acc_sh.at[pl.ds(row0, RPT)])
        plsc.subcore_barrier()

        for k in range(_NB):
            pltpu.async_copy(table_sh.at[s_v.at[k]], rows_v.at[k],
                             sems.at[k])

        def body(j, carry):
            blk0 = j * _NB
            for k in range(_NB):
                blk = blk0 + k
                pltpu.make_async_copy(table_sh.at[s_v.at[k]],
                                      rows_v.at[k], sems.at[k]).wait()
                pltpu.sync_copy(rows_v.at[k], acc_sh.at[d_v.at[blk]],
                                add=True)

                @pl.when(blk + _NB < NBLK)
                def _():
                    pltpu.async_copy(table_sh.at[s_v.at[blk + _NB]],
                                     rows_v.at[k], sems.at[k])
            return carry

        lax.fori_loop(0, NBLK // _NB, body, 0)
        plsc.subcore_barrier()
        pltpu.sync_copy(acc_sh.at[pl.ds(row0, RPT)],
                        out_hbm.at[cid, pl.ds(row0, RPT)])

    return agg


_agg64 = _make_agg(64, jnp.bfloat16)
_agg16 = _make_agg(16, jnp.float32)


_RB = 2000


def _mm_body(x_ref, w_ref, out_ref):
    out_ref[...] = jnp.dot(x_ref[...], w_ref[...],
                           preferred_element_type=jnp.float32)


_mm = pl.pallas_call(
    _mm_body,
    grid=(N // _RB,),
    in_specs=[
        pl.BlockSpec((_RB, 256), lambda i: (i, 0)),
        pl.BlockSpec((256, 64), lambda i: (0, 0)),
    ],
    out_specs=pl.BlockSpec((_RB, 64), lambda i: (i, 0)),
    out_shape=jax.ShapeDtypeStruct((N, 64), jnp.float32),
)


def _scale_body(xw_ref, degt_ref, xws_ref, xwsb_ref, dis_ref):
    d = degt_ref[...]
    deg = d[:, 0:1] + d[:, 1:2] + 1.0
    dis = lax.rsqrt(deg)
    xws = xw_ref[...] * dis
    xws_ref[...] = xws
    xwsb_ref[...] = xws.astype(jnp.bfloat16)
    dis_ref[...] = dis


_scale = pl.pallas_call(
    _scale_body,
    grid=(N // _RB,),
    in_specs=[
        pl.BlockSpec((_RB, 64), lambda i: (i, 0)),
        pl.BlockSpec((_RB, NC), lambda i: (i, 0)),
    ],
    out_specs=[
        pl.BlockSpec((_RB, 64), lambda i: (i, 0)),
        pl.BlockSpec((_RB, 64), lambda i: (i, 0)),
        pl.BlockSpec((_RB, 1), lambda i: (i, 0)),
    ],
    out_shape=[
        jax.ShapeDtypeStruct((N, 64), jnp.float32),
        jax.ShapeDtypeStruct((N, 64), jnp.bfloat16),
        jax.ShapeDtypeStruct((N, 1), jnp.float32),
    ],
)


def _mid_body(a_ref, xws_ref, dis_ref, b1_ref, w2_ref, out_ref):
    d = dis_ref[...]
    acc = (a_ref[0] + a_ref[1]).astype(jnp.float32)
    pre = (acc + xws_ref[...]) * d + b1_ref[...]
    h = jnp.maximum(pre, 0.0)
    out_ref[...] = jnp.dot(h, w2_ref[...],
                           preferred_element_type=jnp.float32) * d


_mid = pl.pallas_call(
    _mid_body,
    grid=(N // _RB,),
    in_specs=[
        pl.BlockSpec((NC, _RB, 64), lambda i: (0, i, 0)),
        pl.BlockSpec((_RB, 64), lambda i: (i, 0)),
        pl.BlockSpec((_RB, 1), lambda i: (i, 0)),
        pl.BlockSpec((1, 64), lambda i: (0, 0)),
        pl.BlockSpec((64, 16), lambda i: (0, 0)),
    ],
    out_specs=pl.BlockSpec((_RB, 16), lambda i: (i, 0)),
    out_shape=jax.ShapeDtypeStruct((N, 16), jnp.float32),
)


def _final_body(q_ref, hw_ref, dis_ref, b2_ref, out_ref):
    logits = (q_ref[0] + q_ref[1] + hw_ref[...]) * dis_ref[...] + b2_ref[...]
    col = lax.broadcasted_iota(jnp.int32, logits.shape, 1)
    valid = col < 5
    masked = jnp.where(valid, logits, -jnp.inf)
    m = jnp.max(masked, axis=1, keepdims=True)
    e = jnp.where(valid, jnp.exp(logits - m), 0.0)
    lse = jnp.log(jnp.sum(e, axis=1, keepdims=True))
    out_ref[...] = (logits - m - lse)[:, :5]


_final = pl.pallas_call(
    _final_body,
    grid=(N // _RB,),
    in_specs=[
        pl.BlockSpec((NC, _RB, 16), lambda i: (0, i, 0)),
        pl.BlockSpec((_RB, 16), lambda i: (i, 0)),
        pl.BlockSpec((_RB, 1), lambda i: (i, 0)),
        pl.BlockSpec((1, 16), lambda i: (0, 0)),
    ],
    out_specs=pl.BlockSpec((_RB, 5), lambda i: (i, 0)),
    out_shape=jax.ShapeDtypeStruct((N, 5), jnp.float32),
)


def kernel(x, edge_index, W1, b1, W2, b2):
    ei = edge_index.astype(jnp.int32)
    ph = N + (jnp.arange(EP - E, dtype=jnp.int32) % NPH)
    srcdst = jnp.concatenate([ei, jnp.stack([ph, ph])], axis=1)
    srcdst = srcdst.reshape(2, NW, NBLK, BLK)
    z640 = jnp.zeros((RPT,), jnp.float32)
    z640_64 = jnp.zeros((RPT, 64), jnp.bfloat16)
    z640_16 = jnp.zeros((RPT, 16), jnp.float32)
    w2p = jnp.pad(W2, ((0, 0), (0, 16 - W2.shape[1])))
    b1r = b1.reshape(1, 64)
    b2p = jnp.pad(b2, (0, 16 - b2.shape[0])).reshape(1, 16)

    deg = _deg_kernel(srcdst, z640)
    xw = _mm(x, W1)
    xws, xwsb, dis = _scale(xw, deg.T)
    a = _agg64(xwsb, srcdst, z640_64)
    hw2s = _mid(a, xws, dis, b1r, w2p)
    q = _agg16(hw2s, srcdst, z640_16)
    return _final(q, hw2s, dis, b2p)


# speedup vs baseline: 34.0530x; 1.0466x over previous
import functools

import jax
import jax.numpy as jnp
from jax import lax
from jax.experimental import pallas as pl
from jax.experimental.pallas import tpu as pltpu
from jax.experimental.pallas import tpu_sc as plsc

N = 10000
NP = 10240
E = 160000
NC, NS = 2, 16
NW = NC * NS
BLK = 128
EPT = 5120
EP = NW * EPT
NBLK = EPT // BLK
RPT = NP // NS
NPH = NP - N

_mesh = plsc.VectorSubcoreMesh(core_axis_name="c", subcore_axis_name="s")
_sc_params = pltpu.CompilerParams(use_tc_tiling_on_sc=False)


@functools.partial(
    pl.kernel,
    out_type=jax.ShapeDtypeStruct((NC, NP), jnp.float32),
    mesh=_mesh,
    scratch_types=[
        pltpu.VMEM((NBLK, BLK), jnp.int32),
        pltpu.VMEM((BLK,), jnp.float32),
        pltpu.VMEM_SHARED((NP,), jnp.float32),
    ],
    compiler_params=_sc_params,
)
def _deg_kernel(srcdst_hbm, z640_hbm, out_hbm, idx_v, ones_v, deg_sh):
    cid = lax.axis_index("c")
    sid = lax.axis_index("s")
    wid = cid * NS + sid
    for j in range(BLK // 16):
        ones_v[pl.ds(16 * j, 16)] = jnp.full((16,), 1.0, jnp.float32)
    row0 = sid * RPT
    pltpu.sync_copy(srcdst_hbm.at[1, wid], idx_v)
    pltpu.sync_copy(z640_hbm, deg_sh.at[pl.ds(row0, RPT)])
    plsc.subcore_barrier()

    def body(i, carry):
        pltpu.sync_copy(ones_v, deg_sh.at[idx_v.at[i]], add=True)
        return carry

    lax.fori_loop(0, NBLK, body, 0)
    plsc.subcore_barrier()
    pltpu.sync_copy(deg_sh.at[pl.ds(row0, RPT)],
                    out_hbm.at[cid, pl.ds(row0, RPT)])


_NB = 8


def _make_agg(width, dtype):
    @functools.partial(
        pl.kernel,
        out_type=jax.ShapeDtypeStruct((NC, NP, width), dtype),
        mesh=_mesh,
        scratch_types=[
            pltpu.VMEM((NBLK, BLK), jnp.int32),
            pltpu.VMEM((NBLK, BLK), jnp.int32),
            pltpu.VMEM((_NB, BLK, width), dtype),
            pltpu.SemaphoreType.DMA((_NB,)),
            pltpu.VMEM_SHARED((NP, width), dtype),
            pltpu.VMEM_SHARED((NP, width), dtype),
        ],
        name=f"gcn_agg{width}",
        compiler_params=_sc_params,
    )
    def agg(table_hbm, srcdst_hbm, z640_hbm, out_hbm,
            s_v, d_v, rows_v, sems, acc_sh, table_sh):
        cid = lax.axis_index("c")
        sid = lax.axis_index("s")
        wid = cid * NS + sid
        row0 = sid * RPT
        pltpu.sync_copy(srcdst_hbm.at[0, wid], s_v)
        pltpu.sync_copy(srcdst_hbm.at[1, wid], d_v)

        @pl.when(sid < NS - 1)
        def _():
            pltpu.sync_copy(table_hbm.at[pl.ds(row0, RPT)],
                            table_sh.at[pl.ds(row0, RPT)])

        @pl.when(sid == NS - 1)
        def _():
            pltpu.sync_copy(table_hbm.at[pl.ds(NS * RPT - RPT, N - (NS - 1) * RPT)],
                            table_sh.at[pl.ds(NS * RPT - RPT, N - (NS - 1) * RPT)])
            pltpu.sync_copy(z640_hbm.at[pl.ds(0, NPH)],
                            table_sh.at[pl.ds(N, NPH)])

        pltpu.sync_copy(z640_hbm, acc_sh.at[pl.ds(row0, RPT)])
        plsc.subcore_barrier()

        for k in range(_NB):
            pltpu.async_copy(table_sh.at[s_v.at[k]], rows_v.at[k],
                             sems.at[k])

        def body(j, carry):
            blk0 = j * _NB
            for k in range(_NB):
                blk = blk0 + k
                pltpu.make_async_copy(table_sh.at[s_v.at[k]],
                                      rows_v.at[k], sems.at[k]).wait()
                pltpu.sync_copy(rows_v.at[k], acc_sh.at[d_v.at[blk]],
                                add=True)

                @pl.when(blk + _NB < NBLK)
                def _():
                    pltpu.async_copy(table_sh.at[s_v.at[blk + _NB]],
                                     rows_v.at[k], sems.at[k])
            return carry

        lax.fori_loop(0, NBLK // _NB, body, 0)
        plsc.subcore_barrier()
        pltpu.sync_copy(acc_sh.at[pl.ds(row0, RPT)],
                        out_hbm.at[cid, pl.ds(row0, RPT)])

    return agg


_agg64 = _make_agg(64, jnp.bfloat16)
_agg16 = _make_agg(16, jnp.bfloat16)


_RB = 2000


def _mm_body(x_ref, w_ref, out_ref):
    out_ref[...] = jnp.dot(x_ref[...], w_ref[...],
                           preferred_element_type=jnp.float32)


_mm = pl.pallas_call(
    _mm_body,
    grid=(N // _RB,),
    in_specs=[
        pl.BlockSpec((_RB, 256), lambda i: (i, 0)),
        pl.BlockSpec((256, 64), lambda i: (0, 0)),
    ],
    out_specs=pl.BlockSpec((_RB, 64), lambda i: (i, 0)),
    out_shape=jax.ShapeDtypeStruct((N, 64), jnp.float32),
)


def _scale_body(xw_ref, degt_ref, xws_ref, xwsb_ref, dis_ref):
    d = degt_ref[...]
    deg = d[:, 0:1] + d[:, 1:2] + 1.0
    dis = lax.rsqrt(deg)
    xws = xw_ref[...] * dis
    xws_ref[...] = xws
    xwsb_ref[...] = xws.astype(jnp.bfloat16)
    dis_ref[...] = dis


_scale = pl.pallas_call(
    _scale_body,
    grid=(N // _RB,),
    in_specs=[
        pl.BlockSpec((_RB, 64), lambda i: (i, 0)),
        pl.BlockSpec((_RB, NC), lambda i: (i, 0)),
    ],
    out_specs=[
        pl.BlockSpec((_RB, 64), lambda i: (i, 0)),
        pl.BlockSpec((_RB, 64), lambda i: (i, 0)),
        pl.BlockSpec((_RB, 1), lambda i: (i, 0)),
    ],
    out_shape=[
        jax.ShapeDtypeStruct((N, 64), jnp.float32),
        jax.ShapeDtypeStruct((N, 64), jnp.bfloat16),
        jax.ShapeDtypeStruct((N, 1), jnp.float32),
    ],
)


def _mid_body(a_ref, xws_ref, dis_ref, b1_ref, w2_ref, out_ref):
    d = dis_ref[...]
    acc = (a_ref[0] + a_ref[1]).astype(jnp.float32)
    pre = (acc + xws_ref[...]) * d + b1_ref[...]
    h = jnp.maximum(pre, 0.0)
    hw = jnp.dot(h, w2_ref[...], preferred_element_type=jnp.float32) * d
    out_ref[...] = hw.astype(jnp.bfloat16)


_mid = pl.pallas_call(
    _mid_body,
    grid=(N // _RB,),
    in_specs=[
        pl.BlockSpec((NC, _RB, 64), lambda i: (0, i, 0)),
        pl.BlockSpec((_RB, 64), lambda i: (i, 0)),
        pl.BlockSpec((_RB, 1), lambda i: (i, 0)),
        pl.BlockSpec((1, 64), lambda i: (0, 0)),
        pl.BlockSpec((64, 16), lambda i: (0, 0)),
    ],
    out_specs=pl.BlockSpec((_RB, 16), lambda i: (i, 0)),
    out_shape=jax.ShapeDtypeStruct((N, 16), jnp.bfloat16),
)


def _final_body(q_ref, hw_ref, dis_ref, b2_ref, out_ref):
    agg = (q_ref[0] + q_ref[1]).astype(jnp.float32) \
        + hw_ref[...].astype(jnp.float32)
    logits = agg * dis_ref[...] + b2_ref[...]
    col = lax.broadcasted_iota(jnp.int32, logits.shape, 1)
    valid = col < 5
    masked = jnp.where(valid, logits, -jnp.inf)
    m = jnp.max(masked, axis=1, keepdims=True)
    e = jnp.where(valid, jnp.exp(logits - m), 0.0)
    lse = jnp.log(jnp.sum(e, axis=1, keepdims=True))
    out_ref[...] = (logits - m - lse)[:, :5]


_final = pl.pallas_call(
    _final_body,
    grid=(N // _RB,),
    in_specs=[
        pl.BlockSpec((NC, _RB, 16), lambda i: (0, i, 0)),
        pl.BlockSpec((_RB, 16), lambda i: (i, 0)),
        pl.BlockSpec((_RB, 1), lambda i: (i, 0)),
        pl.BlockSpec((1, 16), lambda i: (0, 0)),
    ],
    out_specs=pl.BlockSpec((_RB, 5), lambda i: (i, 0)),
    out_shape=jax.ShapeDtypeStruct((N, 5), jnp.float32),
)


def kernel(x, edge_index, W1, b1, W2, b2):
    ei = edge_index.astype(jnp.int32)
    ph = N + (jnp.arange(EP - E, dtype=jnp.int32) % NPH)
    srcdst = jnp.concatenate([ei, jnp.stack([ph, ph])], axis=1)
    srcdst = srcdst.reshape(2, NW, NBLK, BLK)
    z640 = jnp.zeros((RPT,), jnp.float32)
    z640_64 = jnp.zeros((RPT, 64), jnp.bfloat16)
    z640_16 = jnp.zeros((RPT, 16), jnp.bfloat16)
    w2p = jnp.pad(W2, ((0, 0), (0, 16 - W2.shape[1])))
    b1r = b1.reshape(1, 64)
    b2p = jnp.pad(b2, (0, 16 - b2.shape[0])).reshape(1, 16)

    deg = _deg_kernel(srcdst, z640)
    xw = _mm(x, W1)
    xws, xwsb, dis = _scale(xw, deg.T)
    a = _agg64(xwsb, srcdst, z640_64)
    hw2s = _mid(a, xws, dis, b1r, w2p)
    q = _agg16(hw2s, srcdst, z640_16)
    return _final(q, hw2s, dis, b2p)


# speedup vs baseline: 34.4183x; 1.0107x over previous
import functools

import jax
import jax.numpy as jnp
from jax import lax
from jax.experimental import pallas as pl
from jax.experimental.pallas import tpu as pltpu
from jax.experimental.pallas import tpu_sc as plsc

N = 10000
NP = 10240
E = 160000
NC, NS = 2, 16
NW = NC * NS
BLK = 128
EPT = 5120
EP = NW * EPT
NBLK = EPT // BLK
RPT = NP // NS
NPH = NP - N

_mesh = plsc.VectorSubcoreMesh(core_axis_name="c", subcore_axis_name="s")
_sc_params = pltpu.CompilerParams(use_tc_tiling_on_sc=False)


@functools.partial(
    pl.kernel,
    out_type=jax.ShapeDtypeStruct((NC, NP), jnp.float32),
    mesh=_mesh,
    scratch_types=[
        pltpu.VMEM((NBLK, BLK), jnp.int32),
        pltpu.VMEM((BLK,), jnp.float32),
        pltpu.SemaphoreType.DMA,
        pltpu.VMEM_SHARED((NP,), jnp.float32),
    ],
    compiler_params=_sc_params,
)
def _deg_kernel(srcdst_hbm, z640_hbm, out_hbm, idx_v, ones_v, ssem, deg_sh):
    cid = lax.axis_index("c")
    sid = lax.axis_index("s")
    wid = cid * NS + sid
    for j in range(BLK // 16):
        ones_v[pl.ds(16 * j, 16)] = jnp.full((16,), 1.0, jnp.float32)
    row0 = sid * RPT
    pltpu.sync_copy(srcdst_hbm.at[1, wid], idx_v)
    pltpu.sync_copy(z640_hbm, deg_sh.at[pl.ds(row0, RPT)])
    plsc.subcore_barrier()

    def body(j, carry):
        for k in range(8):
            pltpu.async_copy(ones_v, deg_sh.at[idx_v.at[j * 8 + k]],
                             ssem, add=True)
        for k in range(8):
            pltpu.make_async_copy(ones_v, deg_sh.at[idx_v.at[j * 8 + k]],
                                  ssem).wait()
        return carry

    lax.fori_loop(0, NBLK // 8, body, 0)
    plsc.subcore_barrier()
    pltpu.sync_copy(deg_sh.at[pl.ds(row0, RPT)],
                    out_hbm.at[cid, pl.ds(row0, RPT)])


_NB = 8


def _make_agg(width, dtype):
    @functools.partial(
        pl.kernel,
        out_type=jax.ShapeDtypeStruct((NC, NP, width), dtype),
        mesh=_mesh,
        scratch_types=[
            pltpu.VMEM((NBLK, BLK), jnp.int32),
            pltpu.VMEM((NBLK, BLK), jnp.int32),
            pltpu.VMEM((_NB, BLK, width), dtype),
            pltpu.SemaphoreType.DMA((_NB,)),
            pltpu.SemaphoreType.DMA((_NB,)),
            pltpu.VMEM_SHARED((NP, width), dtype),
            pltpu.VMEM_SHARED((NP, width), dtype),
        ],
        name=f"gcn_agg{width}",
        compiler_params=_sc_params,
    )
    def agg(table_hbm, srcdst_hbm, z640_hbm, out_hbm,
            s_v, d_v, rows_v, gsems, ssems, acc_sh, table_sh):
        cid = lax.axis_index("c")
        sid = lax.axis_index("s")
        wid = cid * NS + sid
        row0 = sid * RPT
        pltpu.sync_copy(srcdst_hbm.at[0, wid], s_v)
        pltpu.sync_copy(srcdst_hbm.at[1, wid], d_v)

        @pl.when(sid < NS - 1)
        def _():
            pltpu.sync_copy(table_hbm.at[pl.ds(row0, RPT)],
                            table_sh.at[pl.ds(row0, RPT)])

        @pl.when(sid == NS - 1)
        def _():
            pltpu.sync_copy(table_hbm.at[pl.ds(NS * RPT - RPT, N - (NS - 1) * RPT)],
                            table_sh.at[pl.ds(NS * RPT - RPT, N - (NS - 1) * RPT)])
            pltpu.sync_copy(z640_hbm.at[pl.ds(0, NPH)],
                            table_sh.at[pl.ds(N, NPH)])

        pltpu.sync_copy(z640_hbm, acc_sh.at[pl.ds(row0, RPT)])
        plsc.subcore_barrier()

        for k in range(_NB):
            pltpu.async_copy(table_sh.at[s_v.at[k]], rows_v.at[k],
                             gsems.at[k])

        def body(j, carry):
            blk0 = j * _NB
            for k in range(_NB):
                blk = blk0 + k
                pltpu.make_async_copy(table_sh.at[s_v.at[k]],
                                      rows_v.at[k], gsems.at[k]).wait()
                pltpu.async_copy(rows_v.at[k], acc_sh.at[d_v.at[blk]],
                                 ssems.at[k], add=True)
            for k in range(_NB):
                blk = blk0 + k
                pltpu.make_async_copy(rows_v.at[k], acc_sh.at[d_v.at[blk]],
                                      ssems.at[k]).wait()

                @pl.when(blk + _NB < NBLK)
                def _():
                    pltpu.async_copy(table_sh.at[s_v.at[blk + _NB]],
                                     rows_v.at[k], gsems.at[k])
            return carry

        lax.fori_loop(0, NBLK // _NB, body, 0)
        plsc.subcore_barrier()
        pltpu.sync_copy(acc_sh.at[pl.ds(row0, RPT)],
                        out_hbm.at[cid, pl.ds(row0, RPT)])

    return agg


_agg64 = _make_agg(64, jnp.bfloat16)
_agg16 = _make_agg(16, jnp.bfloat16)


_RB = 2000


def _mm_body(x_ref, w_ref, out_ref):
    out_ref[...] = jnp.dot(x_ref[...], w_ref[...],
                           preferred_element_type=jnp.float32)


_mm = pl.pallas_call(
    _mm_body,
    grid=(N // _RB,),
    in_specs=[
        pl.BlockSpec((_RB, 256), lambda i: (i, 0)),
        pl.BlockSpec((256, 64), lambda i: (0, 0)),
    ],
    out_specs=pl.BlockSpec((_RB, 64), lambda i: (i, 0)),
    out_shape=jax.ShapeDtypeStruct((N, 64), jnp.float32),
)


def _scale_body(xw_ref, degt_ref, xws_ref, xwsb_ref, dis_ref):
    d = degt_ref[...]
    deg = d[:, 0:1] + d[:, 1:2] + 1.0
    dis = lax.rsqrt(deg)
    xws = xw_ref[...] * dis
    xws_ref[...] = xws
    xwsb_ref[...] = xws.astype(jnp.bfloat16)
    dis_ref[...] = dis


_scale = pl.pallas_call(
    _scale_body,
    grid=(N // _RB,),
    in_specs=[
        pl.BlockSpec((_RB, 64), lambda i: (i, 0)),
        pl.BlockSpec((_RB, NC), lambda i: (i, 0)),
    ],
    out_specs=[
        pl.BlockSpec((_RB, 64), lambda i: (i, 0)),
        pl.BlockSpec((_RB, 64), lambda i: (i, 0)),
        pl.BlockSpec((_RB, 1), lambda i: (i, 0)),
    ],
    out_shape=[
        jax.ShapeDtypeStruct((N, 64), jnp.float32),
        jax.ShapeDtypeStruct((N, 64), jnp.bfloat16),
        jax.ShapeDtypeStruct((N, 1), jnp.float32),
    ],
)


def _mid_body(a_ref, xws_ref, dis_ref, b1_ref, w2_ref, out_ref):
    d = dis_ref[...]
    acc = (a_ref[0] + a_ref[1]).astype(jnp.float32)
    pre = (acc + xws_ref[...]) * d + b1_ref[...]
    h = jnp.maximum(pre, 0.0)
    hw = jnp.dot(h, w2_ref[...], preferred_element_type=jnp.float32) * d
    out_ref[...] = hw.astype(jnp.bfloat16)


_mid = pl.pallas_call(
    _mid_body,
    grid=(N // _RB,),
    in_specs=[
        pl.BlockSpec((NC, _RB, 64), lambda i: (0, i, 0)),
        pl.BlockSpec((_RB, 64), lambda i: (i, 0)),
        pl.BlockSpec((_RB, 1), lambda i: (i, 0)),
        pl.BlockSpec((1, 64), lambda i: (0, 0)),
        pl.BlockSpec((64, 16), lambda i: (0, 0)),
    ],
    out_specs=pl.BlockSpec((_RB, 16), lambda i: (i, 0)),
    out_shape=jax.ShapeDtypeStruct((N, 16), jnp.bfloat16),
)


def _final_body(q_ref, hw_ref, dis_ref, b2_ref, out_ref):
    agg = (q_ref[0] + q_ref[1]).astype(jnp.float32) \
        + hw_ref[...].astype(jnp.float32)
    logits = agg * dis_ref[...] + b2_ref[...]
    col = lax.broadcasted_iota(jnp.int32, logits.shape, 1)
    valid = col < 5
    masked = jnp.where(valid, logits, -jnp.inf)
    m = jnp.max(masked, axis=1, keepdims=True)
    e = jnp.where(valid, jnp.exp(logits - m), 0.0)
    lse = jnp.log(jnp.sum(e, axis=1, keepdims=True))
    out_ref[...] = (logits - m - lse)[:, :5]


_final = pl.pallas_call(
    _final_body,
    grid=(N // _RB,),
    in_specs=[
        pl.BlockSpec((NC, _RB, 16), lambda i: (0, i, 0)),
        pl.BlockSpec((_RB, 16), lambda i: (i, 0)),
        pl.BlockSpec((_RB, 1), lambda i: (i, 0)),
        pl.BlockSpec((1, 16), lambda i: (0, 0)),
    ],
    out_specs=pl.BlockSpec((_RB, 5), lambda i: (i, 0)),
    out_shape=jax.ShapeDtypeStruct((N, 5), jnp.float32),
)


def kernel(x, edge_index, W1, b1, W2, b2):
    ei = edge_index.astype(jnp.int32)
    ph = N + (jnp.arange(EP - E, dtype=jnp.int32) % NPH)
    srcdst = jnp.concatenate([ei, jnp.stack([ph, ph])], axis=1)
    srcdst = srcdst.reshape(2, NW, NBLK, BLK)
    z640 = jnp.zeros((RPT,), jnp.float32)
    z640_64 = jnp.zeros((RPT, 64), jnp.bfloat16)
    z640_16 = jnp.zeros((RPT, 16), jnp.bfloat16)
    w2p = jnp.pad(W2, ((0, 0), (0, 16 - W2.shape[1])))
    b1r = b1.reshape(1, 64)
    b2p = jnp.pad(b2, (0, 16 - b2.shape[0])).reshape(1, 16)

    deg = _deg_kernel(srcdst, z640)
    xw = _mm(x, W1)
    xws, xwsb, dis = _scale(xw, deg.T)
    a = _agg64(xwsb, srcdst, z640_64)
    hw2s = _mid(a, xws, dis, b1r, w2p)
    q = _agg16(hw2s, srcdst, z640_16)
    return _final(q, hw2s, dis, b2p)


# speedup vs baseline: 34.8046x; 1.0112x over previous
import functools

import jax
import jax.numpy as jnp
from jax import lax
from jax.experimental import pallas as pl
from jax.experimental.pallas import tpu as pltpu
from jax.experimental.pallas import tpu_sc as plsc

N = 10000
NP = 10240
E = 160000
NC, NS = 2, 16
NW = NC * NS
BLK = 128
EPT = 5120
EP = NW * EPT
NBLK = EPT // BLK
RPT = NP // NS
NPH = NP - N

_mesh = plsc.VectorSubcoreMesh(core_axis_name="c", subcore_axis_name="s")
_sc_params = pltpu.CompilerParams(use_tc_tiling_on_sc=False)


@functools.partial(
    pl.kernel,
    out_type=jax.ShapeDtypeStruct((NC, NP), jnp.float32),
    mesh=_mesh,
    scratch_types=[
        pltpu.VMEM((NBLK, BLK), jnp.int32),
        pltpu.VMEM((BLK,), jnp.float32),
        pltpu.SemaphoreType.DMA,
        pltpu.VMEM_SHARED((NP,), jnp.float32),
    ],
    compiler_params=_sc_params,
)
def _deg_kernel(srcdst_hbm, z640_hbm, out_hbm, idx_v, ones_v, ssem, deg_sh):
    cid = lax.axis_index("c")
    sid = lax.axis_index("s")
    wid = cid * NS + sid
    for j in range(BLK // 16):
        ones_v[pl.ds(16 * j, 16)] = jnp.full((16,), 1.0, jnp.float32)
    row0 = sid * RPT
    pltpu.sync_copy(srcdst_hbm.at[1, wid], idx_v)
    pltpu.sync_copy(z640_hbm, deg_sh.at[pl.ds(row0, RPT)])
    plsc.subcore_barrier()

    def body(j, carry):
        for k in range(8):
            pltpu.async_copy(ones_v, deg_sh.at[idx_v.at[j * 8 + k]],
                             ssem, add=True)
        for k in range(8):
            pltpu.make_async_copy(ones_v, deg_sh.at[idx_v.at[j * 8 + k]],
                                  ssem).wait()
        return carry

    lax.fori_loop(0, NBLK // 8, body, 0)
    plsc.subcore_barrier()
    pltpu.sync_copy(deg_sh.at[pl.ds(row0, RPT)],
                    out_hbm.at[cid, pl.ds(row0, RPT)])


_NB = 8


def _make_agg(width, dtype):
    @functools.partial(
        pl.kernel,
        out_type=jax.ShapeDtypeStruct((NC, NP, width), dtype),
        mesh=_mesh,
        scratch_types=[
            pltpu.VMEM((NBLK, BLK), jnp.int32),
            pltpu.VMEM((NBLK, BLK), jnp.int32),
            pltpu.VMEM((_NB, BLK, width), dtype),
            pltpu.SemaphoreType.DMA((_NB,)),
            pltpu.SemaphoreType.DMA((_NB,)),
            pltpu.VMEM_SHARED((NP, width), dtype),
            pltpu.VMEM_SHARED((NP, width), dtype),
        ],
        name=f"gcn_agg{width}",
        compiler_params=_sc_params,
    )
    def agg(table_hbm, srcdst_hbm, z640_hbm, out_hbm,
            s_v, d_v, rows_v, gsems, ssems, acc_sh, table_sh):
        cid = lax.axis_index("c")
        sid = lax.axis_index("s")
        wid = cid * NS + sid
        row0 = sid * RPT
        pltpu.sync_copy(srcdst_hbm.at[0, wid], s_v)
        pltpu.sync_copy(srcdst_hbm.at[1, wid], d_v)

        @pl.when(sid < NS - 1)
        def _():
            pltpu.sync_copy(table_hbm.at[pl.ds(row0, RPT)],
                            table_sh.at[pl.ds(row0, RPT)])

        @pl.when(sid == NS - 1)
        def _():
            pltpu.sync_copy(table_hbm.at[pl.ds(NS * RPT - RPT, N - (NS - 1) * RPT)],
                            table_sh.at[pl.ds(NS * RPT - RPT, N - (NS - 1) * RPT)])
            pltpu.sync_copy(z640_hbm.at[pl.ds(0, NPH)],
                            table_sh.at[pl.ds(N, NPH)])

        pltpu.sync_copy(z640_hbm, acc_sh.at[pl.ds(row0, RPT)])
        plsc.subcore_barrier()

        for k in range(_NB):
            pltpu.async_copy(table_sh.at[s_v.at[k]], rows_v.at[k],
                             gsems.at[k])

        def body(j, carry):
            blk0 = j * _NB
            for k in range(_NB):
                blk = blk0 + k
                pltpu.make_async_copy(table_sh.at[s_v.at[k]],
                                      rows_v.at[k], gsems.at[k]).wait()
                pltpu.async_copy(rows_v.at[k], acc_sh.at[d_v.at[blk]],
                                 ssems.at[k], add=True)
            for k in range(_NB):
                blk = blk0 + k
                pltpu.make_async_copy(rows_v.at[k], acc_sh.at[d_v.at[blk]],
                                      ssems.at[k]).wait()

                @pl.when(blk + _NB < NBLK)
                def _():
                    pltpu.async_copy(table_sh.at[s_v.at[blk + _NB]],
                                     rows_v.at[k], gsems.at[k])
            return carry

        lax.fori_loop(0, NBLK // _NB, body, 0)
        plsc.subcore_barrier()
        pltpu.sync_copy(acc_sh.at[pl.ds(row0, RPT)],
                        out_hbm.at[cid, pl.ds(row0, RPT)])

    return agg


_agg64 = _make_agg(64, jnp.bfloat16)
_agg16 = _make_agg(16, jnp.bfloat16)


_RB = 2000


def _mm_body(x_ref, w_ref, out_ref):
    out_ref[...] = jnp.dot(x_ref[...], w_ref[...],
                           preferred_element_type=jnp.float32)


_mm = pl.pallas_call(
    _mm_body,
    grid=(N // _RB,),
    in_specs=[
        pl.BlockSpec((_RB, 256), lambda i: (i, 0)),
        pl.BlockSpec((256, 64), lambda i: (0, 0)),
    ],
    out_specs=pl.BlockSpec((_RB, 64), lambda i: (i, 0)),
    out_shape=jax.ShapeDtypeStruct((N, 64), jnp.float32),
)


def _scale_body(xw_ref, degt_ref, xwsb_ref, dis_ref):
    d = degt_ref[...]
    deg = d[:, 0:1] + d[:, 1:2] + 1.0
    dis = lax.rsqrt(deg)
    xwsb_ref[...] = (xw_ref[...] * dis).astype(jnp.bfloat16)
    dis_ref[...] = dis


_scale = pl.pallas_call(
    _scale_body,
    grid=(N // _RB,),
    in_specs=[
        pl.BlockSpec((_RB, 64), lambda i: (i, 0)),
        pl.BlockSpec((_RB, NC), lambda i: (i, 0)),
    ],
    out_specs=[
        pl.BlockSpec((_RB, 64), lambda i: (i, 0)),
        pl.BlockSpec((_RB, 1), lambda i: (i, 0)),
    ],
    out_shape=[
        jax.ShapeDtypeStruct((N, 64), jnp.bfloat16),
        jax.ShapeDtypeStruct((N, 1), jnp.float32),
    ],
)


def _mid_body(a_ref, xws_ref, dis_ref, b1_ref, w2_ref, out_ref):
    d = dis_ref[...]
    acc = (a_ref[0].astype(jnp.float32) + a_ref[1].astype(jnp.float32)
           + xws_ref[...].astype(jnp.float32))
    pre = acc * d + b1_ref[...]
    h = jnp.maximum(pre, 0.0)
    hw = jnp.dot(h, w2_ref[...], preferred_element_type=jnp.float32) * d
    out_ref[...] = hw.astype(jnp.bfloat16)


_mid = pl.pallas_call(
    _mid_body,
    grid=(N // _RB,),
    in_specs=[
        pl.BlockSpec((NC, _RB, 64), lambda i: (0, i, 0)),
        pl.BlockSpec((_RB, 64), lambda i: (i, 0)),
        pl.BlockSpec((_RB, 1), lambda i: (i, 0)),
        pl.BlockSpec((1, 64), lambda i: (0, 0)),
        pl.BlockSpec((64, 16), lambda i: (0, 0)),
    ],
    out_specs=pl.BlockSpec((_RB, 16), lambda i: (i, 0)),
    out_shape=jax.ShapeDtypeStruct((N, 16), jnp.bfloat16),
)


def _final_body(q_ref, hw_ref, dis_ref, b2_ref, out_ref):
    agg = (q_ref[0] + q_ref[1]).astype(jnp.float32) \
        + hw_ref[...].astype(jnp.float32)
    logits = agg * dis_ref[...] + b2_ref[...]
    col = lax.broadcasted_iota(jnp.int32, logits.shape, 1)
    valid = col < 5
    masked = jnp.where(valid, logits, -jnp.inf)
    m = jnp.max(masked, axis=1, keepdims=True)
    e = jnp.where(valid, jnp.exp(logits - m), 0.0)
    lse = jnp.log(jnp.sum(e, axis=1, keepdims=True))
    out_ref[...] = (logits - m - lse)[:, :5]


_final = pl.pallas_call(
    _final_body,
    grid=(N // _RB,),
    in_specs=[
        pl.BlockSpec((NC, _RB, 16), lambda i: (0, i, 0)),
        pl.BlockSpec((_RB, 16), lambda i: (i, 0)),
        pl.BlockSpec((_RB, 1), lambda i: (i, 0)),
        pl.BlockSpec((1, 16), lambda i: (0, 0)),
    ],
    out_specs=pl.BlockSpec((_RB, 5), lambda i: (i, 0)),
    out_shape=jax.ShapeDtypeStruct((N, 5), jnp.float32),
)


def kernel(x, edge_index, W1, b1, W2, b2):
    ei = edge_index.astype(jnp.int32)
    ph = N + (jnp.arange(EP - E, dtype=jnp.int32) % NPH)
    srcdst = jnp.concatenate([ei, jnp.stack([ph, ph])], axis=1)
    srcdst = srcdst.reshape(2, NW, NBLK, BLK)
    z640 = jnp.zeros((RPT,), jnp.float32)
    z640_64 = jnp.zeros((RPT, 64), jnp.bfloat16)
    z640_16 = jnp.zeros((RPT, 16), jnp.bfloat16)
    w2p = jnp.pad(W2, ((0, 0), (0, 16 - W2.shape[1])))
    b1r = b1.reshape(1, 64)
    b2p = jnp.pad(b2, (0, 16 - b2.shape[0])).reshape(1, 16)

    deg = _deg_kernel(srcdst, z640)
    xw = _mm(x, W1)
    xwsb, dis = _scale(xw, deg.T)
    a = _agg64(xwsb, srcdst, z640_64)
    hw2s = _mid(a, xwsb, dis, b1r, w2p)
    q = _agg16(hw2s, srcdst, z640_16)
    return _final(q, hw2s, dis, b2p)


# speedup vs baseline: 35.8571x; 1.0302x over previous
import functools

import jax
import jax.numpy as jnp
from jax import lax
from jax.experimental import pallas as pl
from jax.experimental.pallas import tpu as pltpu
from jax.experimental.pallas import tpu_sc as plsc

N = 10000
NP = 10240
E = 160000
NC, NS = 2, 16
NW = NC * NS
BLK = 128
EPT = 5120
EP = NW * EPT
NBLK = EPT // BLK
RPT = NP // NS
NPH = NP - N

_mesh = plsc.VectorSubcoreMesh(core_axis_name="c", subcore_axis_name="s")
_sc_params = pltpu.CompilerParams(use_tc_tiling_on_sc=False)


@functools.partial(
    pl.kernel,
    out_type=jax.ShapeDtypeStruct((NC, NP), jnp.float32),
    mesh=_mesh,
    scratch_types=[
        pltpu.VMEM((NBLK, BLK), jnp.int32),
        pltpu.VMEM((BLK,), jnp.float32),
        pltpu.SemaphoreType.DMA,
        pltpu.VMEM_SHARED((NP,), jnp.float32),
    ],
    compiler_params=_sc_params,
)
def _deg_kernel(srcdst_hbm, z640_hbm, out_hbm, idx_v, ones_v, ssem, deg_sh):
    cid = lax.axis_index("c")
    sid = lax.axis_index("s")
    wid = cid * NS + sid
    for j in range(BLK // 16):
        ones_v[pl.ds(16 * j, 16)] = jnp.full((16,), 1.0, jnp.float32)
    row0 = sid * RPT
    pltpu.sync_copy(srcdst_hbm.at[1, wid], idx_v)
    pltpu.sync_copy(z640_hbm, deg_sh.at[pl.ds(row0, RPT)])
    plsc.subcore_barrier()

    def body(j, carry):
        for k in range(8):
            pltpu.async_copy(ones_v, deg_sh.at[idx_v.at[j * 8 + k]],
                             ssem, add=True)
        for k in range(8):
            pltpu.make_async_copy(ones_v, deg_sh.at[idx_v.at[j * 8 + k]],
                                  ssem).wait()
        return carry

    lax.fori_loop(0, NBLK // 8, body, 0)
    plsc.subcore_barrier()
    pltpu.sync_copy(deg_sh.at[pl.ds(row0, RPT)],
                    out_hbm.at[cid, pl.ds(row0, RPT)])


_NB = 8


def _make_agg(width, dtype):
    @functools.partial(
        pl.kernel,
        out_type=jax.ShapeDtypeStruct((NC, NP, width), dtype),
        mesh=_mesh,
        scratch_types=[
            pltpu.VMEM((NBLK, BLK), jnp.int32),
            pltpu.VMEM((NBLK, BLK), jnp.int32),
            pltpu.VMEM((_NB, BLK, width), dtype),
            pltpu.SemaphoreType.DMA((_NB,)),
            pltpu.SemaphoreType.DMA((_NB,)),
            pltpu.VMEM_SHARED((NP, width), dtype),
            pltpu.VMEM_SHARED((NP, width), dtype),
        ],
        name=f"gcn_agg{width}",
        compiler_params=_sc_params,
    )
    def agg(table_hbm, srcdst_hbm, z640_hbm, out_hbm,
            s_v, d_v, rows_v, gsems, ssems, acc_sh, table_sh):
        cid = lax.axis_index("c")
        sid = lax.axis_index("s")
        wid = cid * NS + sid
        row0 = sid * RPT
        pltpu.async_copy(srcdst_hbm.at[0, wid], s_v, gsems.at[0])
        pltpu.async_copy(srcdst_hbm.at[1, wid], d_v, gsems.at[1])
        pltpu.async_copy(z640_hbm, acc_sh.at[pl.ds(row0, RPT)], ssems.at[0])

        @pl.when(sid < NS - 1)
        def _():
            pltpu.async_copy(table_hbm.at[pl.ds(row0, RPT)],
                             table_sh.at[pl.ds(row0, RPT)], ssems.at[1])
            pltpu.make_async_copy(table_hbm.at[pl.ds(row0, RPT)],
                                  table_sh.at[pl.ds(row0, RPT)],
                                  ssems.at[1]).wait()

        @pl.when(sid == NS - 1)
        def _():
            pltpu.async_copy(
                table_hbm.at[pl.ds(NS * RPT - RPT, N - (NS - 1) * RPT)],
                table_sh.at[pl.ds(NS * RPT - RPT, N - (NS - 1) * RPT)],
                ssems.at[1])
            pltpu.async_copy(z640_hbm.at[pl.ds(0, NPH)],
                             table_sh.at[pl.ds(N, NPH)], ssems.at[2])
            pltpu.make_async_copy(
                table_hbm.at[pl.ds(NS * RPT - RPT, N - (NS - 1) * RPT)],
                table_sh.at[pl.ds(NS * RPT - RPT, N - (NS - 1) * RPT)],
                ssems.at[1]).wait()
            pltpu.make_async_copy(z640_hbm.at[pl.ds(0, NPH)],
                                  table_sh.at[pl.ds(N, NPH)],
                                  ssems.at[2]).wait()

        pltpu.make_async_copy(srcdst_hbm.at[0, wid], s_v, gsems.at[0]).wait()
        pltpu.make_async_copy(srcdst_hbm.at[1, wid], d_v, gsems.at[1]).wait()
        pltpu.make_async_copy(z640_hbm, acc_sh.at[pl.ds(row0, RPT)],
                              ssems.at[0]).wait()
        plsc.subcore_barrier()

        for k in range(_NB):
            pltpu.async_copy(table_sh.at[s_v.at[k]], rows_v.at[k],
                             gsems.at[k])

        def body(j, carry):
            blk0 = j * _NB
            for k in range(_NB):
                blk = blk0 + k
                pltpu.make_async_copy(table_sh.at[s_v.at[k]],
                                      rows_v.at[k], gsems.at[k]).wait()
                pltpu.async_copy(rows_v.at[k], acc_sh.at[d_v.at[blk]],
                                 ssems.at[k], add=True)
            for k in range(_NB):
                blk = blk0 + k
                pltpu.make_async_copy(rows_v.at[k], acc_sh.at[d_v.at[blk]],
                                      ssems.at[k]).wait()

                @pl.when(blk + _NB < NBLK)
                def _():
                    pltpu.async_copy(table_sh.at[s_v.at[blk + _NB]],
                                     rows_v.at[k], gsems.at[k])
            return carry

        lax.fori_loop(0, NBLK // _NB, body, 0)
        plsc.subcore_barrier()
        pltpu.sync_copy(acc_sh.at[pl.ds(row0, RPT)],
                        out_hbm.at[cid, pl.ds(row0, RPT)])

    return agg


_agg64 = _make_agg(64, jnp.bfloat16)
_agg16 = _make_agg(16, jnp.bfloat16)


_RB = 2000


def _mm_body(x_ref, w_ref, out_ref):
    out_ref[...] = jnp.dot(x_ref[...], w_ref[...],
                           preferred_element_type=jnp.float32)


_mm = pl.pallas_call(
    _mm_body,
    grid=(N // _RB,),
    in_specs=[
        pl.BlockSpec((_RB, 256), lambda i: (i, 0)),
        pl.BlockSpec((256, 64), lambda i: (0, 0)),
    ],
    out_specs=pl.BlockSpec((_RB, 64), lambda i: (i, 0)),
    out_shape=jax.ShapeDtypeStruct((N, 64), jnp.float32),
)


def _scale_body(xw_ref, degt_ref, xwsb_ref, dis_ref):
    d = degt_ref[...]
    deg = d[:, 0:1] + d[:, 1:2] + 1.0
    dis = lax.rsqrt(deg)
    xwsb_ref[...] = (xw_ref[...] * dis).astype(jnp.bfloat16)
    dis_ref[...] = dis


_scale = pl.pallas_call(
    _scale_body,
    grid=(N // _RB,),
    in_specs=[
        pl.BlockSpec((_RB, 64), lambda i: (i, 0)),
        pl.BlockSpec((_RB, NC), lambda i: (i, 0)),
    ],
    out_specs=[
        pl.BlockSpec((_RB, 64), lambda i: (i, 0)),
        pl.BlockSpec((_RB, 1), lambda i: (i, 0)),
    ],
    out_shape=[
        jax.ShapeDtypeStruct((N, 64), jnp.bfloat16),
        jax.ShapeDtypeStruct((N, 1), jnp.float32),
    ],
)


def _mid_body(a_ref, xws_ref, dis_ref, b1_ref, w2_ref, out_ref):
    d = dis_ref[...]
    acc = (a_ref[0].astype(jnp.float32) + a_ref[1].astype(jnp.float32)
           + xws_ref[...].astype(jnp.float32))
    pre = acc * d + b1_ref[...]
    h = jnp.maximum(pre, 0.0)
    hw = jnp.dot(h, w2_ref[...], preferred_element_type=jnp.float32) * d
    out_ref[...] = hw.astype(jnp.bfloat16)


_mid = pl.pallas_call(
    _mid_body,
    grid=(N // _RB,),
    in_specs=[
        pl.BlockSpec((NC, _RB, 64), lambda i: (0, i, 0)),
        pl.BlockSpec((_RB, 64), lambda i: (i, 0)),
        pl.BlockSpec((_RB, 1), lambda i: (i, 0)),
        pl.BlockSpec((1, 64), lambda i: (0, 0)),
        pl.BlockSpec((64, 16), lambda i: (0, 0)),
    ],
    out_specs=pl.BlockSpec((_RB, 16), lambda i: (i, 0)),
    out_shape=jax.ShapeDtypeStruct((N, 16), jnp.bfloat16),
)


def _final_body(q_ref, hw_ref, dis_ref, b2_ref, out_ref):
    agg = (q_ref[0] + q_ref[1]).astype(jnp.float32) \
        + hw_ref[...].astype(jnp.float32)
    logits = agg * dis_ref[...] + b2_ref[...]
    col = lax.broadcasted_iota(jnp.int32, logits.shape, 1)
    valid = col < 5
    masked = jnp.where(valid, logits, -jnp.inf)
    m = jnp.max(masked, axis=1, keepdims=True)
    e = jnp.where(valid, jnp.exp(logits - m), 0.0)
    lse = jnp.log(jnp.sum(e, axis=1, keepdims=True))
    out_ref[...] = (logits - m - lse)[:, :5]


_final = pl.pallas_call(
    _final_body,
    grid=(N // _RB,),
    in_specs=[
        pl.BlockSpec((NC, _RB, 16), lambda i: (0, i, 0)),
        pl.BlockSpec((_RB, 16), lambda i: (i, 0)),
        pl.BlockSpec((_RB, 1), lambda i: (i, 0)),
        pl.BlockSpec((1, 16), lambda i: (0, 0)),
    ],
    out_specs=pl.BlockSpec((_RB, 5), lambda i: (i, 0)),
    out_shape=jax.ShapeDtypeStruct((N, 5), jnp.float32),
)


def kernel(x, edge_index, W1, b1, W2, b2):
    ei = edge_index.astype(jnp.int32)
    ph = N + (jnp.arange(EP - E, dtype=jnp.int32) % NPH)
    srcdst = jnp.concatenate([ei, jnp.stack([ph, ph])], axis=1)
    srcdst = srcdst.reshape(2, NW, NBLK, BLK)
    z640 = jnp.zeros((RPT,), jnp.float32)
    z640_64 = jnp.zeros((RPT, 64), jnp.bfloat16)
    z640_16 = jnp.zeros((RPT, 16), jnp.bfloat16)
    w2p = jnp.pad(W2, ((0, 0), (0, 16 - W2.shape[1])))
    b1r = b1.reshape(1, 64)
    b2p = jnp.pad(b2, (0, 16 - b2.shape[0])).reshape(1, 16)

    deg = _deg_kernel(srcdst, z640)
    xw = _mm(x, W1)
    xwsb, dis = _scale(xw, deg.T)
    a = _agg64(xwsb, srcdst, z640_64)
    hw2s = _mid(a, xwsb, dis, b1r, w2p)
    q = _agg16(hw2s, srcdst, z640_16)
    return _final(q, hw2s, dis, b2p)
